# baseline (speedup 1.0000x reference)
import functools

import jax
import jax.numpy as jnp
from jax import lax
from jax.experimental import pallas as pl
from jax.experimental.pallas import tpu as pltpu
from jax.experimental.pallas import tpu_sc as plsc

C_IN = 8
H = 16
W = 16
KH = 3
KW = 3
OC = 16
K = 2
HO = H - KH + 1
WO = W - KW + 1
S = HO * WO
N_RF = C_IN * KH * KW
B = 32
T = OC * S * N_RF

NC = 2
NS = 16
NW = 28
S_PER_W = S // NW

XLEN = B * C_IN * H * W
ROWS_OC = S_PER_W * N_RF
CI_OC = ROWS_OC * K
WT_W = 4 * OC * ROWS_OC
OUT_W = S_PER_W * B * OC

_mesh = plsc.VectorSubcoreMesh(core_axis_name="c", subcore_axis_name="s")


@functools.partial(
    pl.kernel,
    mesh=_mesh,
    compiler_params=pltpu.CompilerParams(needs_layout_passes=False),
    out_type=jax.ShapeDtypeStruct((S * B * OC,), jnp.float32),
    scratch_types=[
        pltpu.VMEM((XLEN,), jnp.float32),
        pltpu.VMEM((OC * CI_OC,), jnp.int32),
        pltpu.VMEM((WT_W,), jnp.float32),
        pltpu.VMEM((OUT_W,), jnp.float32),
        pltpu.SemaphoreType.DMA,
    ],
)
def _lutconv_sc(x_hbm, ci_hbm, wt_hbm, out_hbm, x_v, ci_v, wt_v, o_v, sem):
    wid = lax.axis_index("s") * NC + lax.axis_index("c")

    @pl.when(wid < NW)
    def _body():
        copies = [
            pltpu.async_copy(x_hbm, x_v, sem),
            pltpu.async_copy(ci_hbm.at[pl.ds(wid * (OC * CI_OC), OC * CI_OC)],
                             ci_v, sem),
            pltpu.async_copy(wt_hbm.at[pl.ds(wid * WT_W, WT_W)], wt_v, sem),
        ]
        for h in copies:
            h.wait()

        zero = jnp.zeros((OC,), jnp.float32)
        iota = lax.iota(jnp.int32, OC)
        ioc_ci = iota * CI_OC
        ioc_wt = iota * (4 * ROWS_OC)

        BG = 16
        for si in range(S_PER_W):
            o_base0 = si * B * OC
            sw0 = zero

            for bg in range(0, B, BG):
                first = bg == 0

                def rf_body(rf, carry, si=si, bg=bg, first=first):
                    cib = ioc_ci + (si * (N_RF * K) + rf * K)
                    rv = ioc_wt + (si * N_RF + rf)
                    ci0 = plsc.load_gather(ci_v, [cib])
                    ci1 = plsc.load_gather(ci_v, [cib + 1])
                    w0 = plsc.load_gather(wt_v, [rv])
                    w1 = plsc.load_gather(wt_v, [rv + ROWS_OC])
                    w2 = plsc.load_gather(wt_v, [rv + 2 * ROWS_OC])
                    w3 = plsc.load_gather(wt_v, [rv + 3 * ROWS_OC])
                    bb = w2 - w0
                    cc = w1 - w0
                    aa = (w3 + w0) - (w1 + w2)
                    out = []
                    for i in range(BG):
                        off = (bg + i) * (C_IN * H * W)
                        p0 = plsc.load_gather(x_v, [ci0 + off])
                        p1 = plsc.load_gather(x_v, [ci1 + off])
                        out.append(carry[i]
                                   + (p0 * bb + (p1 * cc + (p0 * p1) * aa)))
                    if first:
                        out.append(carry[BG] + w0)
                    return tuple(out)

                init = (zero,) * (BG + 1 if first else BG)
                accs = lax.fori_loop(0, N_RF, rf_body, init)
                if first:
                    sw0 = accs[BG]
                for i in range(BG):
                    o_v[pl.ds(o_base0 + (bg + i) * OC, OC)] = accs[i] + sw0

        pltpu.sync_copy(o_v, out_hbm.at[pl.ds(wid * OUT_W, OUT_W)])


def kernel(x, input_mask, lut_weights):
    xf = x.reshape(-1)
    flat = (input_mask[:, 0] * (H * W) + input_mask[:, 1] * W
            + input_mask[:, 2]).astype(jnp.int32)
    ci_arr = flat.reshape(OC, NW, CI_OC).transpose(1, 0, 2).reshape(-1)
    wt_arr = (lut_weights.reshape(OC, NW, ROWS_OC, 4)
              .transpose(1, 0, 3, 2).reshape(-1))
    out = _lutconv_sc(xf, ci_arr, wt_arr)
    out = out.reshape(S, B, OC)
    return out.transpose(1, 2, 0).reshape(B, OC, HO, WO)


# speedup vs baseline: 1.4869x; 1.4869x over previous
import functools

import jax
import jax.numpy as jnp
from jax import lax
from jax.experimental import pallas as pl
from jax.experimental.pallas import tpu as pltpu
from jax.experimental.pallas import tpu_sc as plsc

C_IN = 8
H = 16
W = 16
KH = 3
KW = 3
OC = 16
K = 2
HO = H - KH + 1
WO = W - KW + 1
S = HO * WO
N_RF = C_IN * KH * KW
B = 32
T = OC * S * N_RF

NC = 2
NS = 16
NW = NC * NS
SH = S // 2
SV = (SH + 15) // 16

XLEN = B * C_IN * H * W
CI_W = SH * N_RF * K
WT_J = SH * N_RF
OUT_W = SH * B

_mesh = plsc.VectorSubcoreMesh(core_axis_name="c", subcore_axis_name="s")


@functools.partial(
    pl.kernel,
    mesh=_mesh,
    compiler_params=pltpu.CompilerParams(needs_layout_passes=False),
    out_type=jax.ShapeDtypeStruct((OC * S * B,), jnp.float32),
    scratch_types=[
        pltpu.VMEM((XLEN,), jnp.float32),
        pltpu.VMEM((CI_W,), jnp.int32),
        pltpu.VMEM((4 * WT_J,), jnp.float32),
        pltpu.VMEM((OUT_W,), jnp.float32),
        pltpu.SemaphoreType.DMA,
    ],
)
def _lutconv_sc(x_hbm, ci_hbm, wt_hbm, out_hbm, x_v, ci_v, wt_v, o_v, sem):
    wid = lax.axis_index("s") * NC + lax.axis_index("c")
    oc = wid // 2
    half = wid % 2

    t0 = oc * (S * N_RF)
    copies = [
        pltpu.async_copy(x_hbm, x_v, sem),
        pltpu.async_copy(ci_hbm.at[pl.ds(t0 * K + half * CI_W, CI_W)],
                         ci_v, sem),
    ]
    for j in range(4):
        copies.append(pltpu.async_copy(
            wt_hbm.at[pl.ds(j * T + t0 + half * WT_J, WT_J)],
            wt_v.at[pl.ds(j * WT_J, WT_J)], sem))
    for h in copies:
        h.wait()

    zero = jnp.zeros((16,), jnp.float32)
    iota = lax.iota(jnp.int32, 16)

    BG = 16
    for sv in range(SV):
        nlanes = min(16, SH - sv * 16)
        mk = None if nlanes == 16 else (iota < nlanes)
        is_ci = (iota + sv * 16) * (N_RF * K)
        is_wt = (iota + sv * 16) * N_RF
        is_ot = (iota + sv * 16) * B
        sw0 = zero

        for bg in range(0, B, BG):
            first = bg == 0

            def rf_body(rf, carry, bg=bg, first=first,
                        is_ci=is_ci, is_wt=is_wt, mk=mk):
                ci0 = plsc.load_gather(ci_v, [is_ci + rf * K], mask=mk)
                ci1 = plsc.load_gather(ci_v, [is_ci + (rf * K + 1)], mask=mk)
                rv = is_wt + rf
                w0 = plsc.load_gather(wt_v, [rv], mask=mk)
                w1 = plsc.load_gather(wt_v, [rv + WT_J], mask=mk)
                w2 = plsc.load_gather(wt_v, [rv + 2 * WT_J], mask=mk)
                w3 = plsc.load_gather(wt_v, [rv + 3 * WT_J], mask=mk)
                bb = w2 - w0
                cc = w1 - w0
                aa = (w3 + w0) - (w1 + w2)
                out = []
                for i in range(BG):
                    off = (bg + i) * (C_IN * H * W)
                    p0 = plsc.load_gather(x_v, [ci0 + off], mask=mk)
                    p1 = plsc.load_gather(x_v, [ci1 + off], mask=mk)
                    out.append(carry[i]
                               + (p0 * bb + (p1 * cc + (p0 * p1) * aa)))
                if first:
                    out.append(carry[BG] + w0)
                return tuple(out)

            init = (zero,) * (BG + 1 if first else BG)
            accs = lax.fori_loop(0, N_RF, rf_body, init)
            if first:
                sw0 = accs[BG]
            for i in range(BG):
                plsc.store_scatter(o_v, [is_ot + (bg + i)], accs[i] + sw0,
                                   mask=mk)

    pltpu.sync_copy(o_v, out_hbm.at[pl.ds(oc * (S * B) + half * OUT_W, OUT_W)])


def kernel(x, input_mask, lut_weights):
    xf = x.reshape(-1)
    flat = (input_mask[:, 0] * (H * W) + input_mask[:, 1] * W
            + input_mask[:, 2]).astype(jnp.int32)
    wt_cols = lut_weights.T.reshape(-1)
    out = _lutconv_sc(xf, flat, wt_cols)
    out = out.reshape(OC, S, B)
    return out.transpose(2, 0, 1).reshape(B, OC, HO, WO)


# speedup vs baseline: 1.6877x; 1.1350x over previous
import functools

import jax
import jax.numpy as jnp
from jax import lax
from jax.experimental import pallas as pl
from jax.experimental.pallas import tpu as pltpu
from jax.experimental.pallas import tpu_sc as plsc

C_IN = 8
H = 16
W = 16
KH = 3
KW = 3
OC = 16
K = 2
HO = H - KH + 1
WO = W - KW + 1
S = HO * WO
N_RF = C_IN * KH * KW
B = 32
T = OC * S * N_RF

NC = 2
NS = 16
NW = NC * NS
SH = S // 2
SV = (SH + 15) // 16

XLEN = B * C_IN * H * W
CI_W = SH * N_RF * K
WT_J = SH * N_RF
OUT_W = SH * B

_mesh = plsc.VectorSubcoreMesh(core_axis_name="c", subcore_axis_name="s")


@functools.partial(
    pl.kernel,
    mesh=_mesh,
    compiler_params=pltpu.CompilerParams(needs_layout_passes=False),
    out_type=jax.ShapeDtypeStruct((OC * S * B,), jnp.float32),
    scratch_types=[
        pltpu.VMEM((XLEN,), jnp.float32),
        pltpu.VMEM((K * WT_J + 16,), jnp.int32),
        pltpu.VMEM((4 * WT_J + 16,), jnp.float32),
        pltpu.VMEM((OUT_W,), jnp.float32),
        pltpu.SemaphoreType.DMA,
    ],
)
def _lutconv_sc(x_hbm, ci_hbm, wt_hbm, out_hbm, x_v, ci_v, wt_v, o_v, sem):
    wid = lax.axis_index("s") * NC + lax.axis_index("c")
    oc = wid // 2
    half = wid % 2

    t0 = oc * (S * N_RF)
    copies = [pltpu.async_copy(ci_hbm.at[pl.ds(t0 * K + half * CI_W, CI_W)],
                               x_v.at[pl.ds(0, CI_W)], sem)]
    for j in range(4):
        copies.append(pltpu.async_copy(
            wt_hbm.at[pl.ds(j * T + t0 + half * WT_J, WT_J)],
            x_v.at[pl.ds(CI_W + j * WT_J, WT_J)], sem))
    for h in copies:
        h.wait()

    zero = jnp.zeros((16,), jnp.float32)
    iota = lax.iota(jnp.int32, 16)

    pat_ci = (iota & 1) * WT_J + (iota >> 1) * SH
    pat_w = iota * SH

    def ci_tr(p, _):
        s = p // 9
        g = p % 9
        vals = x_v[pl.ds(s * (N_RF * K) + g * 16, 16)]
        plsc.store_scatter(ci_v, [pat_ci + (g * 8 * SH + s)],
                           plsc.bitcast(vals, jnp.int32))
        return 0

    lax.fori_loop(0, SH * 9, ci_tr, 0)

    mk8 = iota < 8

    def wt_tr(p, _):
        j = p // (SH * 5)
        r = p % (SH * 5)
        s = r // 5
        g = r % 5
        base = CI_W + j * WT_J + s * N_RF + g * 16
        tgt = pat_w + (j * WT_J + g * 16 * SH + s)
        vals = x_v[pl.ds(base, 16)]
        plsc.store_scatter(wt_v, [tgt], vals,
                           mask=jnp.where(g == 4, mk8, iota < 16))
        return 0

    lax.fori_loop(0, 4 * SH * 5, wt_tr, 0)

    pltpu.async_copy(x_hbm, x_v, sem).wait()

    BG = 16
    for sv in range(SV):
        nlanes = min(16, SH - sv * 16)
        mk = None if nlanes == 16 else (iota < nlanes)
        is_ot = (iota + sv * 16) * B
        sw0 = zero

        for bg in range(0, B, BG):
            first = bg == 0

            def rf_body(rf, carry, sv=sv, bg=bg, first=first, mk=mk):
                rbase = rf * SH + sv * 16
                ci0 = ci_v[pl.ds(rbase, 16)]
                ci1 = ci_v[pl.ds(WT_J + rbase, 16)]
                w0 = wt_v[pl.ds(rbase, 16)]
                w1 = wt_v[pl.ds(WT_J + rbase, 16)]
                w2 = wt_v[pl.ds(2 * WT_J + rbase, 16)]
                w3 = wt_v[pl.ds(3 * WT_J + rbase, 16)]
                bb = w2 - w0
                cc = w1 - w0
                aa = (w3 + w0) - (w1 + w2)
                out = []
                for i in range(BG):
                    off = (bg + i) * (C_IN * H * W)
                    p0 = plsc.load_gather(x_v, [ci0 + off], mask=mk)
                    p1 = plsc.load_gather(x_v, [ci1 + off], mask=mk)
                    out.append(carry[i]
                               + (p0 * (bb + p1 * aa) + p1 * cc))
                if first:
                    out.append(carry[BG] + w0)
                return tuple(out)

            init = (zero,) * (BG + 1 if first else BG)
            accs = lax.fori_loop(0, N_RF, rf_body, init)
            if first:
                sw0 = accs[BG]
            for i in range(BG):
                plsc.store_scatter(o_v, [is_ot + (bg + i)], accs[i] + sw0,
                                   mask=mk)

    pltpu.sync_copy(o_v, out_hbm.at[pl.ds(oc * (S * B) + half * OUT_W, OUT_W)])


def kernel(x, input_mask, lut_weights):
    xf = x.reshape(-1)
    flat = (input_mask[:, 0] * (H * W) + input_mask[:, 1] * W
            + input_mask[:, 2]).astype(jnp.int32)
    flat_bits = lax.bitcast_convert_type(flat, jnp.float32)
    wt_cols = lut_weights.T.reshape(-1)
    out = _lutconv_sc(xf, flat_bits, wt_cols)
    out = out.reshape(OC, S, B)
    return out.transpose(2, 0, 1).reshape(B, OC, HO, WO)


# speedup vs baseline: 1.8379x; 1.0890x over previous
import functools

import jax
import jax.numpy as jnp
from jax import lax
from jax.experimental import pallas as pl
from jax.experimental.pallas import tpu as pltpu
from jax.experimental.pallas import tpu_sc as plsc

C_IN = 8
H = 16
W = 16
KH = 3
KW = 3
OC = 16
K = 2
HO = H - KH + 1
WO = W - KW + 1
S = HO * WO
N_RF = C_IN * KH * KW
B = 32
T = OC * S * N_RF

NC = 2
NS = 16
NW = NC * NS
SH = S // 2
SV = (SH + 15) // 16

XLEN = B * C_IN * H * W
CI_W = SH * N_RF * K
WT_J = SH * N_RF
OUT_W = SH * B

_mesh = plsc.VectorSubcoreMesh(core_axis_name="c", subcore_axis_name="s")


@functools.partial(
    pl.kernel,
    mesh=_mesh,
    compiler_params=pltpu.CompilerParams(needs_layout_passes=False),
    out_type=jax.ShapeDtypeStruct((OC * S * B,), jnp.float32),
    scratch_types=[
        pltpu.VMEM((XLEN,), jnp.float32),
        pltpu.VMEM((K * WT_J + 16,), jnp.int32),
        pltpu.VMEM((4 * WT_J + 16,), jnp.float32),
        pltpu.VMEM((OUT_W,), jnp.float32),
        pltpu.SemaphoreType.DMA,
        pltpu.SemaphoreType.DMA,
    ],
)
def _lutconv_sc(x_hbm, ci_hbm, wt_hbm, out_hbm,
                x_v, ci_v, wt_v, o_v, sem, sem2):
    wid = lax.axis_index("s") * NC + lax.axis_index("c")
    oc = wid // 2
    half = wid % 2

    t0 = oc * (S * N_RF)
    ci_copy = pltpu.async_copy(ci_hbm.at[pl.ds(t0 * K + half * CI_W, CI_W)],
                               x_v.at[pl.ds(0, CI_W)], sem)
    wt_copies = []
    for j in range(4):
        wt_copies.append(pltpu.async_copy(
            wt_hbm.at[pl.ds(j * T + t0 + half * WT_J, WT_J)],
            x_v.at[pl.ds(CI_W + j * WT_J, WT_J)], sem2))
    ci_copy.wait()

    zero = jnp.zeros((16,), jnp.float32)
    iota = lax.iota(jnp.int32, 16)

    pat_ci = (iota & 1) * WT_J + (iota >> 1) * SH
    pat_w = iota * SH

    def ci_tr(p, _):
        s = p // 9
        g = p % 9
        vals = x_v[pl.ds(s * (N_RF * K) + g * 16, 16)]
        plsc.store_scatter(ci_v, [pat_ci + (g * 8 * SH + s)],
                           plsc.bitcast(vals, jnp.int32))
        return 0

    lax.fori_loop(0, SH * 9, ci_tr, 0)

    x1 = pltpu.async_copy(x_hbm.at[pl.ds(0, CI_W)],
                          x_v.at[pl.ds(0, CI_W)], sem)
    for h in wt_copies:
        h.wait()

    mk8 = iota < 8

    def wt_tr(p, _):
        j = p // (SH * 5)
        r = p % (SH * 5)
        s = r // 5
        g = r % 5
        base = CI_W + j * WT_J + s * N_RF + g * 16
        tgt = pat_w + (j * WT_J + g * 16 * SH + s)
        vals = x_v[pl.ds(base, 16)]
        plsc.store_scatter(wt_v, [tgt], vals,
                           mask=jnp.where(g == 4, mk8, iota < 16))
        return 0

    lax.fori_loop(0, 4 * SH * 5, wt_tr, 0)

    x2 = pltpu.async_copy(x_hbm.at[pl.ds(CI_W, XLEN - CI_W)],
                          x_v.at[pl.ds(CI_W, XLEN - CI_W)], sem)
    x1.wait()
    x2.wait()

    BG = 16
    for sv in range(SV):
        nlanes = min(16, SH - sv * 16)
        mk = None if nlanes == 16 else (iota < nlanes)
        is_ot = (iota + sv * 16) * B
        sw0 = zero

        for bg in range(0, B, BG):
            first = bg == 0

            def rf_body(rf, carry, sv=sv, bg=bg, first=first, mk=mk):
                rbase = rf * SH + sv * 16
                ci0 = ci_v[pl.ds(rbase, 16)]
                ci1 = ci_v[pl.ds(WT_J + rbase, 16)]
                w0 = wt_v[pl.ds(rbase, 16)]
                w1 = wt_v[pl.ds(WT_J + rbase, 16)]
                w2 = wt_v[pl.ds(2 * WT_J + rbase, 16)]
                w3 = wt_v[pl.ds(3 * WT_J + rbase, 16)]
                bb = w2 - w0
                cc = w1 - w0
                aa = (w3 + w0) - (w1 + w2)
                out = []
                for i in range(BG):
                    xb = x_v.at[pl.ds((bg + i) * (C_IN * H * W),
                                      C_IN * H * W)]
                    p0 = plsc.load_gather(xb, [ci0], mask=mk)
                    p1 = plsc.load_gather(xb, [ci1], mask=mk)
                    out.append(carry[i]
                               + (p0 * (bb + p1 * aa) + p1 * cc))
                if first:
                    out.append(carry[BG] + w0)
                return tuple(out)

            init = (zero,) * (BG + 1 if first else BG)
            accs = lax.fori_loop(0, N_RF, rf_body, init)
            if first:
                sw0 = accs[BG]
            for i in range(BG):
                plsc.store_scatter(o_v, [is_ot + (bg + i)], accs[i] + sw0,
                                   mask=mk)

    pltpu.sync_copy(o_v, out_hbm.at[pl.ds(oc * (S * B) + half * OUT_W, OUT_W)])


def kernel(x, input_mask, lut_weights):
    xf = x.reshape(-1)
    flat = (input_mask[:, 0] * (H * W) + input_mask[:, 1] * W
            + input_mask[:, 2]).astype(jnp.int32)
    flat_bits = lax.bitcast_convert_type(flat, jnp.float32)
    wt_cols = lut_weights.T.reshape(-1)
    out = _lutconv_sc(xf, flat_bits, wt_cols)
    out = out.reshape(OC, S, B)
    return out.transpose(2, 0, 1).reshape(B, OC, HO, WO)
